# baseline (speedup 1.0000x reference)
import jax
import jax.numpy as jnp
from jax import lax
from jax.experimental import pallas as pl
from jax.experimental.pallas import tpu as pltpu
from jax.experimental.pallas import tpu_sc as plsc

MAXLEN = 200
EMBED_DIM = 64
BATCH = 4096

NC = 2
NS = 16
NW = NC * NS

SEQ_W = BATCH // NW
SEQ_C = 2
NCHUNK = SEQ_W // SEQ_C
NSUB = 5
SUB = MAXLEN // NSUB
NSLOT = 3


def _body(x_hbm, tok_hbm, pos_hbm, out_hbm,
          idx_all, rows0, rows1, rows2,
          psem0, psem1, psem2, gsem0, gsem1, gsem2, osem0, osem1, osem2):
    wid = lax.axis_index("s") * NC + lax.axis_index("c")
    rows = (rows0, rows1, rows2)
    psem = (psem0, psem1, psem2)
    gsem = (gsem0, gsem1, gsem2)
    osem = (osem0, osem1, osem2)

    pltpu.sync_copy(x_hbm.at[pl.ds(wid * SEQ_W, SEQ_W)], idx_all)

    def fire_prep(b):
        for s in range(SEQ_C):
            pltpu.async_copy(pos_hbm, rows[b].at[s], psem[b])

    def wait_prep(b):
        for s in range(SEQ_C):
            pltpu.make_async_copy(pos_hbm, rows[b].at[s], psem[b]).wait()

    def fire_gathers(c, b):
        for s in range(SEQ_C):
            for j in range(NSUB):
                pltpu.async_copy(
                    tok_hbm.at[idx_all.at[c * SEQ_C + s, pl.ds(j * SUB, SUB)]],
                    rows[b].at[s, pl.ds(j * SUB, SUB)],
                    gsem[b],
                    add=True,
                )

    def wait_gathers(b):
        pltpu.make_async_copy(out_hbm.at[pl.ds(0, SEQ_C)], rows[b], gsem[b]).wait()

    def fire_scatter(c, b):
        pltpu.async_copy(rows[b], out_hbm.at[pl.ds(wid * SEQ_W + c * SEQ_C, SEQ_C)], osem[b])

    def wait_scatter(b):
        pltpu.make_async_copy(out_hbm.at[pl.ds(0, SEQ_C)], rows[b], osem[b]).wait()

    def step(c, b, *, do_gather=True, drain_b2=True, do_prep=True):
        b1, b2 = (b + 1) % NSLOT, (b + 2) % NSLOT
        wait_gathers(b)
        fire_scatter(c, b)
        if do_gather:
            wait_prep(b1)
            fire_gathers(c + 1, b1)
        if drain_b2:
            wait_scatter(b2)
        if do_prep:
            fire_prep(b2)

    fire_prep(0)
    fire_prep(1)
    wait_prep(0)
    fire_gathers(0, 0)
    step(0, 0, drain_b2=False)
    step(1, 1)
    step(2, 2)

    @pl.loop(1, (NCHUNK - 4) // NSLOT)
    def _g(g):
        c0 = g * NSLOT
        step(c0 + 0, 0)
        step(c0 + 1, 1)
        step(c0 + 2, 2)

    step(NCHUNK - 4, 0)
    step(NCHUNK - 3, 1)
    step(NCHUNK - 2, 2, do_prep=False)
    step(NCHUNK - 1, 0, do_gather=False, drain_b2=False, do_prep=False)
    wait_scatter(2)
    wait_scatter(0)


@jax.jit
def _run(x, token_table, pos_table):
    mesh = plsc.VectorSubcoreMesh(core_axis_name="c", subcore_axis_name="s")
    f = pl.kernel(
        _body,
        out_type=jax.ShapeDtypeStruct((BATCH, MAXLEN, EMBED_DIM), jnp.float32),
        mesh=mesh,
        scratch_types=(
            [pltpu.VMEM((SEQ_W, MAXLEN), jnp.int32)]
            + [pltpu.VMEM((SEQ_C, MAXLEN, EMBED_DIM), jnp.float32) for _ in range(NSLOT)]
            + [pltpu.SemaphoreType.DMA for _ in range(3 * NSLOT)]
        ),
        compiler_params=pltpu.CompilerParams(use_tc_tiling_on_sc=False),
    )
    return f(x, token_table, pos_table)


def kernel(x, token_table, pos_table):
    return _run(x.astype(jnp.int32), token_table, pos_table)


# speedup vs baseline: 1.2616x; 1.2616x over previous
import jax
import jax.numpy as jnp
from jax import lax
from jax.experimental import pallas as pl
from jax.experimental.pallas import tpu as pltpu
from jax.experimental.pallas import tpu_sc as plsc

MAXLEN = 200
EMBED_DIM = 64
BATCH = 4096

NC = 2
NS = 16
NW = NC * NS

ROWS = BATCH * MAXLEN
PER_W = ROWS // NW
CHUNK = 400
NCHUNK = PER_W // CHUNK
NSUB = 5
SUB = CHUNK // NSUB
NSLOT = 3


def _body(x_hbm, tok_hbm, pos_hbm, out_hbm,
          shared_pos, idx_all, rows0, rows1, rows2,
          psem0, psem1, psem2, gsem0, gsem1, gsem2, osem0, osem1, osem2):
    wid = lax.axis_index("s") * NC + lax.axis_index("c")
    rows = (rows0, rows1, rows2)
    psem = (psem0, psem1, psem2)
    gsem = (gsem0, gsem1, gsem2)
    osem = (osem0, osem1, osem2)

    @pl.when(lax.axis_index("s") == 0)
    def _stage_pos():
        pltpu.sync_copy(pos_hbm, rows0.at[pl.ds(0, MAXLEN)])
        pltpu.sync_copy(rows0.at[pl.ds(0, MAXLEN)], shared_pos.at[pl.ds(0, MAXLEN)])
        pltpu.sync_copy(rows0.at[pl.ds(0, MAXLEN)], shared_pos.at[pl.ds(MAXLEN, MAXLEN)])
    plsc.subcore_barrier()

    pltpu.sync_copy(x_hbm.at[pl.ds(wid * NCHUNK * NSUB, NCHUNK * NSUB)], idx_all)

    def fire_prep(b):
        pltpu.async_copy(shared_pos, rows[b], psem[b])

    def wait_prep(b):
        pltpu.make_async_copy(shared_pos, rows[b], psem[b]).wait()

    def fire_gathers(c, b):
        for j in range(NSUB):
            pltpu.async_copy(
                tok_hbm.at[idx_all.at[c * NSUB + j]],
                rows[b].at[pl.ds(j * SUB, SUB)],
                gsem[b],
                add=True,
            )

    def wait_gathers(b):
        pltpu.make_async_copy(out_hbm.at[pl.ds(0, CHUNK)], rows[b], gsem[b]).wait()

    def fire_scatter(c, b):
        pltpu.async_copy(rows[b], out_hbm.at[pl.ds(wid * PER_W + c * CHUNK, CHUNK)], osem[b])

    def wait_scatter(b):
        pltpu.make_async_copy(out_hbm.at[pl.ds(0, CHUNK)], rows[b], osem[b]).wait()

    def step(c, b, *, do_gather=True, drain_b2=True, do_prep=True):
        b1, b2 = (b + 1) % NSLOT, (b + 2) % NSLOT
        wait_gathers(b)
        fire_scatter(c, b)
        if do_gather:
            wait_prep(b1)
            fire_gathers(c + 1, b1)
        if drain_b2:
            wait_scatter(b2)
        if do_prep:
            fire_prep(b2)

    fire_prep(0)
    fire_prep(1)
    wait_prep(0)
    fire_gathers(0, 0)
    step(0, 0, drain_b2=False)
    step(1, 1)
    step(2, 2)

    @pl.loop(1, (NCHUNK - 4) // NSLOT)
    def _g(g):
        c0 = g * NSLOT
        step(c0 + 0, 0)
        step(c0 + 1, 1)
        step(c0 + 2, 2)

    step(NCHUNK - 4, 0)
    step(NCHUNK - 3, 1)
    step(NCHUNK - 2, 2, do_prep=False)
    step(NCHUNK - 1, 0, do_gather=False, drain_b2=False, do_prep=False)
    wait_scatter(2)
    wait_scatter(0)


@jax.jit
def _run(x, token_table, pos_table):
    x_flat = x.astype(jnp.int32).reshape(ROWS // SUB, SUB)
    mesh = plsc.VectorSubcoreMesh(core_axis_name="c", subcore_axis_name="s")
    f = pl.kernel(
        _body,
        out_type=jax.ShapeDtypeStruct((ROWS, EMBED_DIM), jnp.float32),
        mesh=mesh,
        scratch_types=(
            [pltpu.VMEM_SHARED((CHUNK, EMBED_DIM), jnp.float32)]
            + [pltpu.VMEM((NCHUNK * NSUB, SUB), jnp.int32)]
            + [pltpu.VMEM((CHUNK, EMBED_DIM), jnp.float32) for _ in range(NSLOT)]
            + [pltpu.SemaphoreType.DMA for _ in range(3 * NSLOT)]
        ),
        compiler_params=pltpu.CompilerParams(use_tc_tiling_on_sc=False),
    )
    out = f(x_flat, token_table, pos_table)
    return out.reshape(BATCH, MAXLEN, EMBED_DIM)


def kernel(x, token_table, pos_table):
    return _run(x, token_table, pos_table)
